# baseline (speedup 1.0000x reference)
import jax
import jax.numpy as jnp
from jax import lax
from jax.experimental import pallas as pl
from jax.experimental.pallas import tpu as pltpu
from jax.experimental.pallas import tpu_sc as plsc

NC = 2
NS = 16
NBUF = 2
CHUNKC = 128
BR = 504


def _sc_info(n, e):
    nr4 = 4
    rspan4 = -(-(n + 1) // (nr4 * BR)) * BR
    return rspan4


def _zero_from(buf, rows, acc_sh, base, total):
    done = 0
    while done < total:
        step = min(rows, total - done)
        pltpu.sync_copy(buf.at[pl.ds(0, step)],
                        acc_sh.at[pl.ds(base + done, step)])
        done += step


def _make_sc_agg(table_shape, nvec, npass, rspan, rows_range, chunk, cap):
    width = table_shape[1]
    rows_per_tile = rows_range // NS
    sb = next(d2 for d2 in (160, 128, 80, 64, 40, 32, 20, 16, 10, 8, 5, 4, 2, 1)
              if nvec % d2 == 0)
    mesh = plsc.VectorSubcoreMesh(core_axis_name="c", subcore_axis_name="s")

    def body(*refs):
        (table, src_f, dst_f, out_agg,
         sstage, dstage) = refs[:6]
        i = 6
        bsrc = refs[i:i + npass]
        i += npass
        bdst = refs[i:i + npass]
        i += npass
        bufs = refs[i:i + NBUF]
        i += NBUF
        ibufs = refs[i:i + NBUF]
        i += NBUF
        acc_sh = refs[i]
        i += 1
        sems = refs[i:i + NBUF]

        c = lax.axis_index("c")
        s = lax.axis_index("s")

        los = [(c * npass + p) * rspan for p in range(npass)]

        def bucket_block(b, offs):
            pltpu.sync_copy(src_f.at[s, pl.ds(b * sb, sb)], sstage)
            pltpu.sync_copy(dst_f.at[s, pl.ds(b * sb, sb)], dstage)

            def vec(i2, offs2):
                d = dstage[i2, :]
                sv = sstage[i2, :]
                new = []
                for p in range(npass):
                    lo = los[p]
                    m = jnp.logical_and(d >= lo, d < lo + rspan)
                    off = offs2[p]
                    mi = m.astype(jnp.int32)
                    idxv = off + jnp.cumsum(mi) - 1
                    plsc.store_scatter(bsrc[p], [idxv], sv, mask=m)
                    plsc.store_scatter(bdst[p], [idxv], d - lo, mask=m)
                    new.append(off + jnp.sum(mi))
                return tuple(new)

            return lax.fori_loop(0, sb, vec, offs)

        offs = lax.fori_loop(0, nvec // sb, bucket_block, (0,) * npass)

        pad_src = jnp.zeros((16,), jnp.int32)
        pad_dst = rspan + lax.iota(jnp.int32, 16)
        for p in range(npass):
            off = offs[p]
            for k in range(chunk // 16):
                bsrc[p][pl.ds(off + k * 16, 16)] = pad_src
                bdst[p][pl.ds(off + k * 16, 16)] = pad_dst

        zw = bufs[0].shape[0]
        zeros16 = jnp.zeros((16,), jnp.float32)

        def fill_zeros():
            def zbody(i2, _):
                bufs[0][i2 // (width // 16),
                        pl.ds((i2 % (width // 16)) * 16, 16)] = zeros16
                return 0

            lax.fori_loop(0, zw * (width // 16), zbody, 0)

        fill_zeros()
        _zero_from(bufs[0], zw, acc_sh, s * rows_per_tile, rows_per_tile)
        plsc.subcore_barrier()

        def wait_g(kk):
            pltpu.make_async_copy(
                table.at[pl.ds(0, chunk)], bufs[kk], sems[kk]).wait()

        def wait_s(kk):
            pltpu.make_async_copy(
                bufs[kk], acc_sh.at[ibufs[kk]], sems[kk]).wait()

        for p in range(npass):
            nch = (offs[p] + chunk - 1) // chunk

            def issue_g(ci, kk):
                pltpu.async_copy(
                    table.at[bsrc[p].at[pl.ds(ci * chunk, chunk)]],
                    bufs[kk], sems[kk])

            for kk in range(NBUF):
                @pl.when(kk < nch)
                def _():
                    issue_g(kk, kk)

            def loop(j2, _):
                a = NBUF * j2
                for base in (0, NBUF // 2):
                    for k in range(NBUF // 2):
                        kk = base + k

                        @pl.when(a + kk < nch)
                        def _():
                            wait_g(kk)
                            for t in range(chunk // 16):
                                ibufs[kk][pl.ds(t * 16, 16)] = (
                                    bdst[p][pl.ds((a + kk) * chunk + t * 16, 16)])
                            pltpu.async_copy(
                                bufs[kk], acc_sh.at[ibufs[kk]],
                                sems[kk], add=True)
                    for k in range(NBUF // 2):
                        kk = base + k

                        @pl.when(a + kk < nch)
                        def _():
                            wait_s(kk)

                        @pl.when(a + NBUF + kk < nch)
                        def _():
                            issue_g(a + NBUF + kk, kk)
                return 0

            lax.fori_loop(0, (nch + NBUF - 1) // NBUF, loop, 0)
            plsc.subcore_barrier()

            rows = pl.ds(s * rows_per_tile, rows_per_tile)
            pltpu.sync_copy(acc_sh.at[rows], out_agg.at[c * npass + p, rows])
            if p + 1 < npass:
                fill_zeros()
                _zero_from(bufs[0], zw, acc_sh, s * rows_per_tile,
                           rows_per_tile)
                plsc.subcore_barrier()

    return body, mesh


def _sc_agg(table, src_f, dst_f, npass, rspan, chunk):
    nvec = src_f.shape[1]
    rows_range = rspan + 16
    rows_range = -(-rows_range // NS) * NS
    cap = (nvec * 16 // chunk + 1) * chunk + chunk
    body, mesh = _make_sc_agg(table.shape, nvec, npass, rspan, rows_range,
                              chunk, cap)
    width = table.shape[1]
    scratch = [
        pltpu.VMEM((160, 16), jnp.int32),
        pltpu.VMEM((160, 16), jnp.int32),
    ]
    scratch += [pltpu.VMEM((cap,), jnp.int32) for _ in range(2 * npass)]
    scratch += [pltpu.VMEM((chunk, width), jnp.float32) for _ in range(NBUF)]
    scratch += [pltpu.VMEM((chunk,), jnp.int32) for _ in range(NBUF)]
    scratch.append(pltpu.VMEM_SHARED((rows_range, width), jnp.float32))
    scratch += [pltpu.SemaphoreType.DMA for _ in range(NBUF)]
    f = pl.kernel(
        body,
        out_type=jax.ShapeDtypeStruct((NC * npass, rows_range, width),
                                      jnp.float32),
        mesh=mesh,
        scratch_types=scratch,
        compiler_params=pltpu.CompilerParams(
            use_tc_tiling_on_sc=False, needs_layout_passes=False),
        name="sc_sage_rng%d" % npass,
    )
    return f(table, src_f, dst_f)


def _make_sc_cnt(n_acc, ch):
    rows_per_tile = n_acc // NS
    mesh = plsc.VectorSubcoreMesh(core_axis_name="c", subcore_axis_name="s")

    def body(dst_r, out_cnt, dst_v, ones_v, zc_v, cnt_sh, sem_c):
        c = lax.axis_index("c")
        s = lax.axis_index("s")
        w = c * NS + s

        pltpu.sync_copy(dst_r.at[w], dst_v)
        zeros16 = jnp.zeros((16,), jnp.float32)
        ones16 = jnp.ones((16,), jnp.float32)

        def fill(k, _):
            ones_v[k, pl.ds(0, 16)] = ones16
            zc_v[k % zc_v.shape[0], pl.ds(0, 16)] = zeros16
            return 0

        lax.fori_loop(0, CHUNKC, fill, 0)
        _zero_from(zc_v, zc_v.shape[0], cnt_sh, s * rows_per_tile,
                   rows_per_tile)
        plsc.subcore_barrier()

        def loop(j, _):
            pltpu.async_copy(ones_v, cnt_sh.at[dst_v.at[j]], sem_c, add=True)
            return 0

        lax.fori_loop(0, ch, loop, 0)

        def drain(j, _):
            pltpu.make_async_copy(
                ones_v, cnt_sh.at[dst_v.at[0]], sem_c).wait()
            return 0

        lax.fori_loop(0, ch, drain, 0)
        plsc.subcore_barrier()
        rows = pl.ds(s * rows_per_tile, rows_per_tile)
        pltpu.sync_copy(cnt_sh.at[rows], out_cnt.at[c, rows])

    return body, mesh


def _sc_cnt(dst_r, n_acc):
    ch = dst_r.shape[1]
    body, mesh = _make_sc_cnt(n_acc, ch)
    f = pl.kernel(
        body,
        out_type=jax.ShapeDtypeStruct((NC, n_acc, 16), jnp.float32),
        mesh=mesh,
        scratch_types=[
            pltpu.VMEM(dst_r.shape[1:], jnp.int32),
            pltpu.VMEM((CHUNKC, 16), jnp.float32),
            pltpu.VMEM((CHUNKC, 16), jnp.float32),
            pltpu.VMEM_SHARED((n_acc, 16), jnp.float32),
            pltpu.SemaphoreType.DMA,
        ],
        compiler_params=pltpu.CompilerParams(use_tc_tiling_on_sc=False),
        name="sc_sage_cnt",
    )
    return f(dst_r)


def _tc1_body(agg, cnt, xb, w1l, w1r, b1, w2l, w2r, b2, p_out, r_out):
    rc = 1.0 / jnp.maximum(cnt[0, :, 0:1] + cnt[1, :, 0:1], 1.0)
    mean = agg[0] * rc
    h = jnp.dot(mean, w1l[...], preferred_element_type=jnp.float32)
    h += jnp.dot(xb[...], w1r[...], preferred_element_type=jnp.float32)
    h = jnp.maximum(h + b1[...], 0.0)
    p_out[...] = jnp.dot(h, w2l[...], preferred_element_type=jnp.float32)
    r_out[...] = jnp.dot(h, w2r[...], preferred_element_type=jnp.float32) + b2[...]


def _tc3_body(agg, cnt, r, out):
    rc = 1.0 / jnp.maximum(cnt[0, :, 0:1] + cnt[1, :, 0:1], 1.0)
    out[...] = agg[0] * rc + r[...]


def kernel(x, edge_index, W1l, W1r, b1, W2l, W2r, b2):
    n, d = x.shape
    e = edge_index.shape[1]
    h_dim = W1l.shape[1]
    c_dim = W2l.shape[1]

    rspan4 = _sc_info(n, e)
    rspan2 = 2 * rspan4
    n_cnt = 4 * rspan4

    emul = NC * NS * CHUNKC
    e_pad = ((e + emul - 1) // emul) * emul
    src = edge_index[0]
    dst = edge_index[1]
    if e_pad != e:
        pad_dst = n + jnp.arange(e_pad - e, dtype=jnp.int32) % min(
            64, 4 * rspan4 - n)
        src = jnp.concatenate([src, jnp.zeros((e_pad - e,), jnp.int32)])
        dst = jnp.concatenate([dst, pad_dst])

    src_f = src.reshape(NS, e_pad // NS // 16, 16)
    dst_f = dst.reshape(NS, e_pad // NS // 16, 16)
    dst_r2 = dst.reshape(NC * NS, e_pad // (NC * NS) // CHUNKC, CHUNKC)

    cnt = _sc_cnt(dst_r2, n_cnt)
    agg = _sc_agg(x, src_f, dst_f, npass=2, rspan=rspan4, chunk=64)

    gb = -(-n // BR)
    bpr4 = rspan4 // BR
    bpr2 = rspan2 // BR
    row = lambda i: (i, 0)
    full = lambda i: (0, 0)
    p, r = pl.pallas_call(
        _tc1_body,
        grid=(gb,),
        in_specs=[
            pl.BlockSpec((1, BR, d), lambda i: (i // bpr4, i % bpr4, 0)),
            pl.BlockSpec((NC, BR, 16), lambda i: (0, i, 0)),
            pl.BlockSpec((BR, d), row),
            pl.BlockSpec((d, h_dim), full),
            pl.BlockSpec((d, h_dim), full),
            pl.BlockSpec((1, h_dim), full),
            pl.BlockSpec((h_dim, c_dim), full),
            pl.BlockSpec((h_dim, c_dim), full),
            pl.BlockSpec((1, c_dim), full),
        ],
        out_specs=[pl.BlockSpec((BR, c_dim), row), pl.BlockSpec((BR, c_dim), row)],
        out_shape=[
            jax.ShapeDtypeStruct((n, c_dim), jnp.float32),
            jax.ShapeDtypeStruct((n, c_dim), jnp.float32),
        ],
    )(agg, cnt, x, W1l, W1r, b1.reshape(1, -1), W2l, W2r, b2.reshape(1, -1))

    agg2 = _sc_agg(p, src_f, dst_f, npass=1, rspan=rspan2, chunk=128)

    out = pl.pallas_call(
        _tc3_body,
        grid=(gb,),
        in_specs=[
            pl.BlockSpec((1, BR, c_dim), lambda i: (i // bpr2, i % bpr2, 0)),
            pl.BlockSpec((NC, BR, 16), lambda i: (0, i, 0)),
            pl.BlockSpec((BR, c_dim), row),
        ],
        out_specs=pl.BlockSpec((BR, c_dim), row),
        out_shape=jax.ShapeDtypeStruct((n, c_dim), jnp.float32),
    )(agg2, cnt, r)
    return out


# speedup vs baseline: 1.4612x; 1.4612x over previous
import jax
import jax.numpy as jnp
from jax import lax
from jax.experimental import pallas as pl
from jax.experimental.pallas import tpu as pltpu
from jax.experimental.pallas import tpu_sc as plsc

NC = 2
NS = 16
CHUNK = 128
PANEL = 64
ZROWS = 63
NBUF = 8


def _zero_vmem(ref, rows, width):
    zeros16 = jnp.zeros((16,), jnp.float32)

    def body(i, _):
        r = i // (width // 16)
        c = (i % (width // 16)) * 16
        ref[r, pl.ds(c, 16)] = zeros16
        return 0

    lax.fori_loop(0, rows * (width // 16), body, 0)


def _make_sc_agg(n_acc, ch, npass):
    rows_per_tile = n_acc // NS
    assert rows_per_tile % ZROWS == 0 and ch % NBUF == 0
    mesh = plsc.VectorSubcoreMesh(core_axis_name="c", subcore_axis_name="s")

    def body(*refs):
        tables, src_r, dst_r, out_agg = refs[:4]
        i = 4
        src_v, dst_v = refs[i:i + 2]
        i += 2
        bufs = refs[i:i + NBUF]
        i += NBUF
        zbuf, acc_sh = refs[i:i + 2]
        i += 2
        sems = refs[i:i + NBUF]

        c = lax.axis_index("c")
        s = lax.axis_index("s")

        pltpu.sync_copy(src_r.at[s], src_v)
        pltpu.sync_copy(dst_r.at[s], dst_v)

        _zero_vmem(zbuf, ZROWS, PANEL)

        def zero_acc():
            for k in range(rows_per_tile // ZROWS):
                pltpu.sync_copy(
                    zbuf, acc_sh.at[pl.ds(s * rows_per_tile + k * ZROWS, ZROWS)])

        zero_acc()
        plsc.subcore_barrier()

        def wait_g(kk):
            pltpu.make_async_copy(
                tables.at[0].at[pl.ds(0, CHUNK)], bufs[kk], sems[kk]).wait()

        def wait_s(kk):
            pltpu.make_async_copy(
                bufs[kk], acc_sh.at[dst_v.at[0]], sems[kk]).wait()

        for p in range(npass):
            q = c * npass + p

            def issue_g(chunk_i, kk):
                pltpu.async_copy(
                    tables.at[q].at[src_v.at[chunk_i]], bufs[kk], sems[kk])

            for kk in range(NBUF):
                issue_g(kk, kk)

            def loop(j2, _):
                a = NBUF * j2
                for base in (0, NBUF // 2):
                    for k in range(NBUF // 2):
                        kk = base + k
                        wait_g(kk)
                        pltpu.async_copy(
                            bufs[kk], acc_sh.at[dst_v.at[a + kk]], sems[kk],
                            add=True)
                    for k in range(NBUF // 2):
                        kk = base + k
                        pre = a + NBUF + kk
                        wait_s(kk)

                        @pl.when(pre < ch)
                        def _():
                            issue_g(pre, kk)
                return 0

            lax.fori_loop(0, ch // NBUF, loop, 0)
            plsc.subcore_barrier()

            rows = pl.ds(s * rows_per_tile, rows_per_tile)
            pltpu.sync_copy(acc_sh.at[rows], out_agg.at[c * npass + p, rows])
            if p + 1 < npass:
                zero_acc()
                plsc.subcore_barrier()

    return body, mesh


def _sc_agg(tables, src_r, dst_r, n_acc):
    npass = tables.shape[0] // NC
    ch = src_r.shape[1]
    body, mesh = _make_sc_agg(n_acc, ch, npass)
    scratch = [
        pltpu.VMEM(src_r.shape[1:], jnp.int32),
        pltpu.VMEM(dst_r.shape[1:], jnp.int32),
    ]
    scratch += [pltpu.VMEM((CHUNK, PANEL), jnp.float32) for _ in range(NBUF)]
    scratch += [
        pltpu.VMEM((ZROWS, PANEL), jnp.float32),
        pltpu.VMEM_SHARED((n_acc, PANEL), jnp.float32),
    ]
    scratch += [pltpu.SemaphoreType.DMA for _ in range(NBUF)]
    f = pl.kernel(
        body,
        out_type=jax.ShapeDtypeStruct((NC * npass, n_acc, PANEL), jnp.float32),
        mesh=mesh,
        scratch_types=scratch,
        compiler_params=pltpu.CompilerParams(use_tc_tiling_on_sc=False),
        name="sc_sage_agg%d" % npass,
    )
    return f(tables, src_r, dst_r)


def _make_sc_cnt(n_acc, ch):
    rows_per_tile = n_acc // NS
    assert rows_per_tile % ZROWS == 0
    mesh = plsc.VectorSubcoreMesh(core_axis_name="c", subcore_axis_name="s")

    def body(dst_r, out_cnt, dst_v, ones_v, zc_v, cnt_sh, sem_c):
        c = lax.axis_index("c")
        s = lax.axis_index("s")
        w = c * NS + s

        pltpu.sync_copy(dst_r.at[w], dst_v)
        _zero_vmem(zc_v, ZROWS, 16)
        ones16 = jnp.ones((16,), jnp.float32)

        def fill_ones(k, _):
            ones_v[k, pl.ds(0, 16)] = ones16
            return 0

        lax.fori_loop(0, CHUNK, fill_ones, 0)
        for k in range(rows_per_tile // ZROWS):
            pltpu.sync_copy(
                zc_v, cnt_sh.at[pl.ds(s * rows_per_tile + k * ZROWS, ZROWS)])
        plsc.subcore_barrier()

        def loop(j, _):
            pltpu.async_copy(ones_v, cnt_sh.at[dst_v.at[j]], sem_c, add=True)
            return 0

        lax.fori_loop(0, ch, loop, 0)

        def drain(j, _):
            pltpu.make_async_copy(
                ones_v, cnt_sh.at[dst_v.at[0]], sem_c).wait()
            return 0

        lax.fori_loop(0, ch, drain, 0)
        plsc.subcore_barrier()
        rows = pl.ds(s * rows_per_tile, rows_per_tile)
        pltpu.sync_copy(cnt_sh.at[rows], out_cnt.at[c, rows])

    return body, mesh


def _sc_cnt(dst_r, n_acc):
    ch = dst_r.shape[1]
    body, mesh = _make_sc_cnt(n_acc, ch)
    f = pl.kernel(
        body,
        out_type=jax.ShapeDtypeStruct((NC, n_acc, 16), jnp.float32),
        mesh=mesh,
        scratch_types=[
            pltpu.VMEM(dst_r.shape[1:], jnp.int32),
            pltpu.VMEM((CHUNK, 16), jnp.float32),
            pltpu.VMEM((ZROWS, 16), jnp.float32),
            pltpu.VMEM_SHARED((n_acc, 16), jnp.float32),
            pltpu.SemaphoreType.DMA,
        ],
        compiler_params=pltpu.CompilerParams(use_tc_tiling_on_sc=False),
        name="sc_sage_cnt",
    )
    return f(dst_r)


def _tc1_body(agg, cnt, xb, w1l, w1r, b1, w2l, w2r, b2, p_out, r_out):
    rc = 1.0 / jnp.maximum(cnt[0, :, 0:1] + cnt[1, :, 0:1], 1.0)
    mean = jnp.concatenate([agg[q] for q in range(agg.shape[0])], axis=1) * rc
    h = jnp.dot(mean, w1l[...], preferred_element_type=jnp.float32)
    h += jnp.dot(xb[...], w1r[...], preferred_element_type=jnp.float32)
    h = jnp.maximum(h + b1[...], 0.0)
    pp = jnp.dot(h, w2l[...], preferred_element_type=jnp.float32)
    for q in range(p_out.shape[0]):
        p_out[q] = pp[:, q * PANEL:(q + 1) * PANEL]
    r_out[...] = jnp.dot(h, w2r[...], preferred_element_type=jnp.float32) + b2[...]


def _tc3_body(agg, cnt, r, out):
    rc = 1.0 / jnp.maximum(cnt[0, :, 0:1] + cnt[1, :, 0:1], 1.0)
    mean = jnp.concatenate([agg[q] for q in range(agg.shape[0])], axis=1) * rc
    out[...] = mean + r[...]


def kernel(x, edge_index, W1l, W1r, b1, W2l, W2r, b2):
    n, d = x.shape
    e = edge_index.shape[1]
    h_dim = W1l.shape[1]
    c_dim = W2l.shape[1]

    n_acc = ((n + NS * ZROWS) // (NS * ZROWS)) * (NS * ZROWS)

    emul = NS * CHUNK * NBUF
    e_pad = ((e + emul - 1) // emul) * emul
    src = edge_index[0]
    dst = edge_index[1]
    if e_pad != e:
        src = jnp.concatenate([src, jnp.zeros((e_pad - e,), jnp.int32)])
        dst = jnp.concatenate([dst, jnp.full((e_pad - e,), n, jnp.int32)])

    src_r = src.reshape(NS, e_pad // NS // CHUNK, CHUNK)
    dst_r = dst.reshape(NS, e_pad // NS // CHUNK, CHUNK)
    dst_r2 = dst.reshape(NC * NS, e_pad // (NC * NS) // CHUNK, CHUNK)

    cnt = _sc_cnt(dst_r2, n_acc)
    xq = x.reshape(n, d // PANEL, PANEL).transpose(1, 0, 2)
    agg = _sc_agg(xq, src_r, dst_r, n_acc)

    br = 512
    gb = (n + br - 1) // br
    row = lambda i: (i, 0)
    row3 = lambda i: (0, i, 0)
    full = lambda i: (0, 0)
    nq1 = d // PANEL
    p, r = pl.pallas_call(
        _tc1_body,
        grid=(gb,),
        in_specs=[
            pl.BlockSpec((nq1, br, PANEL), row3),
            pl.BlockSpec((NC, br, 16), row3),
            pl.BlockSpec((br, d), row),
            pl.BlockSpec((d, h_dim), full),
            pl.BlockSpec((d, h_dim), full),
            pl.BlockSpec((1, h_dim), full),
            pl.BlockSpec((h_dim, c_dim), full),
            pl.BlockSpec((h_dim, c_dim), full),
            pl.BlockSpec((1, c_dim), full),
        ],
        out_specs=[
            pl.BlockSpec((c_dim // PANEL, br, PANEL), row3),
            pl.BlockSpec((br, c_dim), row),
        ],
        out_shape=[
            jax.ShapeDtypeStruct((c_dim // PANEL, n, PANEL), jnp.float32),
            jax.ShapeDtypeStruct((n, c_dim), jnp.float32),
        ],
    )(agg, cnt, x, W1l, W1r, b1.reshape(1, -1), W2l, W2r, b2.reshape(1, -1))

    agg2 = _sc_agg(p, src_r, dst_r, n_acc)

    out = pl.pallas_call(
        _tc3_body,
        grid=(gb,),
        in_specs=[
            pl.BlockSpec((c_dim // PANEL, br, PANEL), row3),
            pl.BlockSpec((NC, br, 16), row3),
            pl.BlockSpec((br, c_dim), row),
        ],
        out_specs=pl.BlockSpec((br, c_dim), row),
        out_shape=jax.ShapeDtypeStruct((n, c_dim), jnp.float32),
    )(agg2, cnt, r)
    return out
